# baseline (speedup 1.0000x reference)
import jax
import jax.numpy as jnp
from jax.experimental import pallas as pl
from jax.experimental.pallas import tpu as pltpu

_SEQ = 2048
_K = 6
_NCHUNK = 128
_QB = 512
_NC = 4

def _body(attn_ref, ws_ref, win_ref, out_ref, bufs_ref, sems_ref):
    del ws_ref, out_ref
    for s in range(_K):
        pltpu.make_async_copy(attn_ref.at[s], bufs_ref.at[s], sems_ref.at[s]).start()

    def step(i, acc):
        slot = jax.lax.rem(i, _K)
        pltpu.make_async_copy(attn_ref.at[i], bufs_ref.at[slot], sems_ref.at[slot]).wait()
        psum = jnp.sum(bufs_ref[slot], axis=0, keepdims=True)
        @pl.when(i + _K < _NCHUNK)
        def _pf():
            pltpu.make_async_copy(attn_ref.at[i + _K], bufs_ref.at[slot], sems_ref.at[slot]).start()
        acc = acc + psum
        is_last = jax.lax.rem(i, _NC) == _NC - 1
        @pl.when(is_last)
        def _fin():
            h = jax.lax.div(i, _NC)
            k = jax.lax.broadcasted_iota(jnp.int32, (_SEQ, 64), 0)
            w = jax.lax.broadcasted_iota(jnp.int32, (_SEQ, 64), 1)
            gmat = ((k >= 4) & (k < 2020) & ((k - 4) // 32 == w)).astype(jnp.float32)
            win_ref[pl.ds(h, 1), :] = jnp.dot(acc, gmat, preferred_element_type=jnp.float32)
        return jnp.where(is_last, 0.0, acc)

    jax.lax.fori_loop(0, _NCHUNK, step, jnp.zeros((1, _SEQ), jnp.float32))

def kernel(past_key_values, attn_score_cache, window_scores):
    attn_flat = attn_score_cache.reshape(_NCHUNK, _QB, _SEQ)
    ws_flat = window_scores.reshape(2880000)
    win, out = pl.pallas_call(
        _body,
        in_specs=[pl.BlockSpec(memory_space=pltpu.MemorySpace.HBM),
                  pl.BlockSpec(memory_space=pltpu.MemorySpace.HBM)],
        out_specs=[pl.BlockSpec(memory_space=pltpu.MemorySpace.VMEM),
                   pl.BlockSpec(memory_space=pltpu.MemorySpace.HBM)],
        out_shape=[jax.ShapeDtypeStruct((32, 64), jnp.float32),
                   jax.ShapeDtypeStruct((2880000,), jnp.float32)],
        input_output_aliases={1: 1},
        scratch_shapes=[
            pltpu.VMEM((_K, _QB, _SEQ), jnp.float32),
            pltpu.SemaphoreType.DMA((_K,)),
        ],
    )(attn_flat, ws_flat)
    win63 = win[:, :63]
    idx = jnp.arange(63, dtype=jnp.float32)
    ws = out.reshape(32, 30000, 3)
    ws = ws.at[:, :63, 0].set(win63)
    ws = ws.at[:, :63, 1].set(idx[None, :])
    ws = ws.at[:, :63, 2].set(idx[None, :])
    return ws
